# Initial kernel scaffold; baseline (speedup 1.0000x reference)
import jax
import jax.numpy as jnp
from jax.experimental import pallas as pl


def kernel(x, all_indices, tables, importance):
    raise NotImplementedError("write your pallas kernel here")


# speedup vs baseline: 7.2079x; 7.2079x over previous
import math

import jax
import jax.numpy as jnp
from jax import lax
from jax.experimental import pallas as pl
from jax.experimental.pallas import tpu as pltpu
from jax.experimental.pallas import tpu_sc as plsc

VOCAB = 100000
N_EMBD = 64
BUCKET = 8191
NUM_TABLES = 3
N_TOKENS = 1024 * 200

NUM_CORES = 2
NUM_SUBCORES = 16
LANES = 16
NW = NUM_CORES * NUM_SUBCORES
TOK_PER_W = N_TOKENS // NW
CHUNK = 128
NCHUNK = TOK_PER_W // CHUNK
SCALE = math.sqrt(N_EMBD)


def _splat(v):
    return jnp.full((LANES,), v, jnp.int32)


def _make_lookup():
    mesh = plsc.VectorSubcoreMesh(core_axis_name="c", subcore_axis_name="s")

    def body(x_hbm, idx0, idx1, idx2, w0c, w1c, w2c, tab_hbm, out_hbm,
             x_v, idx_v, w_v, rows_v, out_v, sem):
        wid = lax.axis_index("s") * NUM_CORES + lax.axis_index("c")

        def chunk_body(k, carry):
            base = wid * TOK_PER_W + k * CHUNK
            pltpu.sync_copy(x_hbm.at[pl.ds(base, CHUNK)], x_v)
            h = []
            for i, src in enumerate((idx0, idx1, idx2)):
                h.append(pltpu.async_copy(src.at[x_v], idx_v.at[i], sem))
            for i, src in enumerate((w0c, w1c, w2c)):
                h.append(pltpu.async_copy(src.at[x_v], w_v.at[pl.ds(i * CHUNK, CHUNK)], sem))
            for hh in h:
                hh.wait()
            h = []
            for i in range(NUM_TABLES):
                h.append(pltpu.async_copy(tab_hbm.at[idx_v.at[i]], rows_v.at[i], sem))
            for hh in h:
                hh.wait()

            def tok_body(t, carry2):
                w0 = plsc.load_gather(w_v, [_splat(t)]) * SCALE
                w1 = plsc.load_gather(w_v, [_splat(CHUNK + t)]) * SCALE
                w2 = plsc.load_gather(w_v, [_splat(2 * CHUNK + t)]) * SCALE
                for q in range(N_EMBD // LANES):
                    sl = pl.ds(q * LANES, LANES)
                    acc = (w0 * rows_v[0, t, sl]
                           + w1 * rows_v[1, t, sl]
                           + w2 * rows_v[2, t, sl])
                    out_v[t, sl] = acc
                return carry2

            lax.fori_loop(0, CHUNK, tok_body, 0)
            pltpu.sync_copy(out_v, out_hbm.at[pl.ds(base, CHUNK)])
            return carry

        lax.fori_loop(0, NCHUNK, chunk_body, 0)

    return pl.kernel(
        body,
        out_type=jax.ShapeDtypeStruct((N_TOKENS, N_EMBD), jnp.float32),
        mesh=mesh,
        compiler_params=pltpu.CompilerParams(
            needs_layout_passes=False, use_tc_tiling_on_sc=False),
        scratch_types=[
            pltpu.VMEM((CHUNK,), jnp.int32),
            pltpu.VMEM((NUM_TABLES, CHUNK), jnp.int32),
            pltpu.VMEM((NUM_TABLES * CHUNK,), jnp.float32),
            pltpu.VMEM((NUM_TABLES, CHUNK, N_EMBD), jnp.float32),
            pltpu.VMEM((CHUNK, N_EMBD), jnp.float32),
            pltpu.SemaphoreType.DMA,
        ],
    )


def kernel(x, all_indices, tables, importance):
    x_flat = x.reshape(-1)
    offs = jnp.arange(NUM_TABLES, dtype=jnp.int32) * BUCKET
    idx_t = (all_indices + offs[None, :]).T
    imp_t = importance.T
    tab = tables.reshape(NUM_TABLES * BUCKET, N_EMBD)
    lookup = _make_lookup()
    out = lookup(x_flat, idx_t[0], idx_t[1], idx_t[2],
                 imp_t[0], imp_t[1], imp_t[2], tab)
    return out.reshape(x.shape + (N_EMBD,))


# speedup vs baseline: 11.6774x; 1.6201x over previous
import math

import jax
import jax.numpy as jnp
from jax import lax
from jax.experimental import pallas as pl
from jax.experimental.pallas import tpu as pltpu
from jax.experimental.pallas import tpu_sc as plsc

VOCAB = 100000
N_EMBD = 64
BUCKET = 8191
NUM_TABLES = 3
N_TOKENS = 1024 * 200

NUM_CORES = 2
NUM_SUBCORES = 16
LANES = 16
NW = NUM_CORES * NUM_SUBCORES
TOK_PER_W = N_TOKENS // NW
CHUNK = 128
NCHUNK = TOK_PER_W // CHUNK
SCALE = math.sqrt(N_EMBD)


def _splat(v):
    return jnp.full((LANES,), v, jnp.int32)


def _make_lookup():
    mesh = plsc.VectorSubcoreMesh(core_axis_name="c", subcore_axis_name="s")

    def body(x_hbm, idx0, idx1, idx2, w0c, w1c, w2c, tab_hbm, out_hbm,
             x_v, idx_v, w_v, rows_v, out_v, meta_sem, rows_sem, out_sem):
        wid = lax.axis_index("s") * NUM_CORES + lax.axis_index("c")

        def stage_a(k, p):
            base = wid * TOK_PER_W + k * CHUNK
            pltpu.sync_copy(x_hbm.at[pl.ds(base, CHUNK)], x_v)
            hs = []
            for i, src in enumerate((idx0, idx1, idx2)):
                hs.append(pltpu.async_copy(src.at[x_v], idx_v.at[p, i], meta_sem))
            for i, src in enumerate((w0c, w1c, w2c)):
                hs.append(pltpu.async_copy(
                    src.at[x_v], w_v.at[p, pl.ds(i * CHUNK, CHUNK)], meta_sem))
            for h in hs:
                h.wait()
            for i in range(NUM_TABLES):
                pltpu.async_copy(tab_hbm.at[idx_v.at[p, i]], rows_v.at[p, i],
                                 rows_sem.at[p])

        def wait_rows(p):
            for i in range(NUM_TABLES):
                pltpu.make_async_copy(tab_hbm.at[idx_v.at[p, i]], rows_v.at[p, i],
                                      rows_sem.at[p]).wait()

        def drain_out(p):
            pltpu.make_async_copy(out_v.at[p], out_hbm.at[pl.ds(0, CHUNK)],
                                  out_sem.at[p]).wait()

        def compute(k, p):
            def tok_body(tt, carry2):
                for u in range(2):
                    t = tt * 2 + u
                    w0 = plsc.load_gather(w_v, [_splat(p), _splat(t)]) * SCALE
                    w1 = plsc.load_gather(w_v, [_splat(p), _splat(CHUNK + t)]) * SCALE
                    w2 = plsc.load_gather(w_v, [_splat(p), _splat(2 * CHUNK + t)]) * SCALE
                    for q in range(N_EMBD // LANES):
                        sl = pl.ds(q * LANES, LANES)
                        acc = (w0 * rows_v[p, 0, t, sl]
                               + w1 * rows_v[p, 1, t, sl]
                               + w2 * rows_v[p, 2, t, sl])
                        out_v[p, t, sl] = acc
                return carry2

            lax.fori_loop(0, CHUNK // 2, tok_body, 0)
            base = wid * TOK_PER_W + k * CHUNK
            pltpu.async_copy(out_v.at[p], out_hbm.at[pl.ds(base, CHUNK)],
                             out_sem.at[p])

        stage_a(0, 0)

        def chunk_pair(kk, carry):
            for p in range(2):
                k = kk * 2 + p

                @pl.when(k + 1 < NCHUNK)
                def _():
                    stage_a(k + 1, 1 - p)

                wait_rows(p)

                @pl.when(k >= 2)
                def _():
                    drain_out(p)

                compute(k, p)
            return carry

        lax.fori_loop(0, NCHUNK // 2, chunk_pair, 0)
        drain_out(0)
        drain_out(1)

    return pl.kernel(
        body,
        out_type=jax.ShapeDtypeStruct((N_TOKENS, N_EMBD), jnp.float32),
        mesh=mesh,
        compiler_params=pltpu.CompilerParams(
            needs_layout_passes=False, use_tc_tiling_on_sc=False),
        scratch_types=[
            pltpu.VMEM((CHUNK,), jnp.int32),
            pltpu.VMEM((2, NUM_TABLES, CHUNK), jnp.int32),
            pltpu.VMEM((2, NUM_TABLES * CHUNK), jnp.float32),
            pltpu.VMEM((2, NUM_TABLES, CHUNK, N_EMBD), jnp.float32),
            pltpu.VMEM((2, CHUNK, N_EMBD), jnp.float32),
            pltpu.SemaphoreType.DMA,
            pltpu.SemaphoreType.DMA((2,)),
            pltpu.SemaphoreType.DMA((2,)),
        ],
    )


def kernel(x, all_indices, tables, importance):
    x_flat = x.reshape(-1)
    offs = jnp.arange(NUM_TABLES, dtype=jnp.int32) * BUCKET
    idx_t = (all_indices + offs[None, :]).T
    imp_t = importance.T
    tab = tables.reshape(NUM_TABLES * BUCKET, N_EMBD)
    lookup = _make_lookup()
    out = lookup(x_flat, idx_t[0], idx_t[1], idx_t[2],
                 imp_t[0], imp_t[1], imp_t[2], tab)
    return out.reshape(x.shape + (N_EMBD,))
